# baseline (speedup 1.0000x reference)
import jax
import jax.numpy as jnp
from jax import lax
from jax.experimental import pallas as pl
from jax.experimental.pallas import tpu as pltpu
from jax.experimental.pallas import tpu_sc as plsc

N_NODES = 10000
RANK = 256
N_EDGES = 160000
N_RELS = 100
HALF = 128
NSUB = 16
CH = 40
E_PER_TILE = N_EDGES // NSUB
N_CHUNKS = E_PER_TILE // CH
N_PAD = 10240
ROWS_PER_TILE = N_PAD // NSUB
RCH = 32
PQ_HALF = 2 * N_RELS * HALF
QOFF = N_RELS * HALF
SLOPE = (1.0 / 8.0 + 1.0 / 3.0) / 2.0

_GDN = lax.GatherDimensionNumbers(
    offset_dims=(), collapsed_slice_dims=(0,), start_index_map=(0,))


def _vperm(v, idx):
    return lax.gather(v, idx[:, None], _GDN, (1,),
                      mode=lax.GatherScatterMode.PROMISE_IN_BOUNDS)


def _sc_body(xh, srcb, dst, et, pqf, normf, out,
             pq_v, norm_v, outbuf,
             xb0, xb1, xb2,
             sv0, sv1, sv2, sv3, sv4, sv5,
             dv0, dv1, dv2, dv3, dv4, dv5,
             ev0, ev1, ev2, ev3, ev4, ev5,
             acc,
             gsem0, gsem1, gsem2, ssem0, ssem1, ssem2,
             isem0, isem1, isem2, isem3, isem4, isem5):
    xbufs = (xb0, xb1, xb2)
    srcv = (sv0, sv1, sv2, sv3, sv4, sv5)
    dstv = (dv0, dv1, dv2, dv3, dv4, dv5)
    etv = (ev0, ev1, ev2, ev3, ev4, ev5)
    gsem = (gsem0, gsem1, gsem2)
    ssem = (ssem0, ssem1, ssem2)
    isem = (isem0, isem1, isem2, isem3, isem4, isem5)

    c = lax.axis_index("c")
    s = lax.axis_index("s")
    iota = lax.iota(jnp.int32, 16)
    swap_iota = iota - 2 * (iota % 2) + 1

    pltpu.sync_copy(pqf.at[pl.ds(c * PQ_HALF, PQ_HALF)], pq_v)
    r0 = s * ROWS_PER_TILE
    pltpu.sync_copy(normf.at[pl.ds(r0, ROWS_PER_TILE)],
                    norm_v.at[pl.ds(0, ROWS_PER_TILE)])

    zero = jnp.zeros((16,), jnp.float32)

    @pl.loop(0, RCH)
    def _zero_rows(i):
        row = outbuf.at[i]
        for t in range(HALF // 16):
            row[pl.ds(t * 16, 16)] = zero

    for kk in range(ROWS_PER_TILE // RCH):
        pltpu.sync_copy(outbuf, acc.at[pl.ds(r0 + kk * RCH, RCH)])
    plsc.subcore_barrier()

    e0 = s * E_PER_TILE

    def idx_triple(j, k):
        base = e0 + k * CH
        return ((srcb.at[pl.ds(c * N_EDGES + base, CH)], srcv[j], isem[j]),
                (dst.at[pl.ds(base, CH)], dstv[j], isem[j]),
                (et.at[pl.ds(base, CH)], etv[j].at[pl.ds(0, CH)], isem[j]))

    def idx_copy(j, k):
        for a, b, sm in idx_triple(j, k):
            pltpu.async_copy(a, b, sm)

    def idx_wait(j, k):
        for a, b, sm in idx_triple(j, k):
            pltpu.make_async_copy(a, b, sm).wait()

    def gather_start(r, j):
        pltpu.async_copy(xh.at[srcv[j]], xbufs[r], gsem[r])

    def gather_wait(r, j):
        pltpu.make_async_copy(xh.at[srcv[j]], xbufs[r], gsem[r]).wait()

    def scatter_start(r, j):
        pltpu.async_copy(xbufs[r], acc.at[dstv[j]], ssem[r], add=True)

    def scatter_wait(r, j):
        pltpu.make_async_copy(xbufs[r], acc.at[dstv[j]], ssem[r]).wait()

    def compute(r, j):
        xb = xbufs[r]
        etr = etv[j]

        @plsc.parallel_loop(0, CH, unroll=4)
        def _edge(e):
            xr = xb.at[e]
            pbase = etr[pl.ds(e, 16)][0]
            for t in range(HALF // 16):
                o = t * 16
                xv = xr[pl.ds(o, 16)]
                xs = _vperm(xv, swap_iota)
                pv = pq_v[pl.ds(pbase + o, 16)]
                qv = pq_v[pl.ds(pbase + (QOFF + o), 16)]
                xr[pl.ds(o, 16)] = xv * pv + xs * qv

    def body(k, u, wait_prev=True, do_icopy=True, do_gather=True):
        r = u % 3
        gather_wait(r, u)
        compute(r, u)
        scatter_start(r, u)
        if wait_prev:
            scatter_wait((u + 2) % 3, (u + 5) % 6)
        if do_icopy:
            idx_copy((u + 4) % 6, k + 4)
        if do_gather:
            idx_wait((u + 2) % 6, k + 2)
            gather_start((u + 2) % 3, (u + 2) % 6)

    scope_edges = jax.named_scope("edge_phase")
    scope_edges.__enter__()
    for j in range(4):
        idx_copy(j, j)
    idx_wait(0, 0)
    gather_start(0, 0)
    idx_wait(1, 1)
    gather_start(1, 1)

    body(0, 0, wait_prev=False)
    for k in range(1, 6):
        body(k, k)

    @pl.loop(0, (N_CHUNKS - 10) // 6)
    def _main(kk):
        kb = 6 + kk * 6
        for u in range(6):
            body(kb + u, u)

    for k in range(N_CHUNKS - 4, N_CHUNKS):
        body(k, k % 6, do_icopy=False, do_gather=(k + 2 < N_CHUNKS))

    scatter_wait((N_CHUNKS - 1) % 3, (N_CHUNKS - 1) % 6)
    scope_edges.__exit__(None, None, None)
    plsc.subcore_barrier()

    scope_fin = jax.named_scope("finalize_phase")
    scope_fin.__enter__()
    for kk in range(ROWS_PER_TILE // RCH):
        rbase = r0 + kk * RCH
        pltpu.sync_copy(acc.at[pl.ds(rbase, RCH)], outbuf)

        @pl.loop(0, RCH)
        def _post(i):
            row = outbuf.at[i]
            nrm = norm_v[pl.ds(kk * RCH + i, 16)][0]
            for t in range(HALF // 16):
                o = t * 16
                v = row[pl.ds(o, 16)] * nrm
                row[pl.ds(o, 16)] = jnp.where(v >= 0.0, v, v * SLOPE)

        pltpu.sync_copy(outbuf, out.at[pl.ds(rbase, RCH), pl.ds(c * HALF, HALF)])
    scope_fin.__exit__(None, None, None)


def kernel(x, norm, weight, edge_index, edge_type):
    x = x.astype(jnp.float32)
    src = edge_index[0].astype(jnp.int32)
    dst = edge_index[1].astype(jnp.int32)
    et = edge_type.astype(jnp.int32) * HALF

    w4 = weight.astype(jnp.float32).reshape(N_RELS, RANK // 2, 2, 2)
    p = jnp.stack([w4[:, :, 0, 0], w4[:, :, 1, 1]], -1).reshape(N_RELS, RANK)
    q = jnp.stack([w4[:, :, 1, 0], w4[:, :, 0, 1]], -1).reshape(N_RELS, RANK)
    ph = p.reshape(N_RELS, 2, HALF).transpose(1, 0, 2)
    qh = q.reshape(N_RELS, 2, HALF).transpose(1, 0, 2)
    pqf = jnp.concatenate([ph, qh], axis=1).reshape(-1)

    xh = x.reshape(N_NODES, 2, HALF).transpose(1, 0, 2).reshape(2 * N_NODES, HALF)
    srcb = jnp.concatenate([src, src + N_NODES])
    normf = jnp.pad(norm.astype(jnp.float32).reshape(-1), (0, N_PAD - N_NODES))

    mesh = plsc.VectorSubcoreMesh(core_axis_name="c", subcore_axis_name="s")
    dma = pltpu.SemaphoreType.DMA
    run = pl.kernel(
        _sc_body,
        out_type=jax.ShapeDtypeStruct((N_PAD, RANK), jnp.float32),
        mesh=mesh,
        scratch_types=(
            [pltpu.VMEM((PQ_HALF,), jnp.float32),
             pltpu.VMEM((ROWS_PER_TILE + 16,), jnp.float32),
             pltpu.VMEM((RCH, HALF), jnp.float32)]
            + [pltpu.VMEM((CH, HALF), jnp.float32)] * 3
            + [pltpu.VMEM((CH,), jnp.int32)] * 6
            + [pltpu.VMEM((CH,), jnp.int32)] * 6
            + [pltpu.VMEM((CH + 16,), jnp.int32)] * 6
            + [pltpu.VMEM_SHARED((N_PAD, HALF), jnp.float32)]
            + [dma] * 12
        ),
    )
    return run(xh, srcb, dst, et, pqf, normf)[:N_NODES]


# speedup vs baseline: 1.0188x; 1.0188x over previous
import jax
import jax.numpy as jnp
from jax import lax
from jax.experimental import pallas as pl
from jax.experimental.pallas import tpu as pltpu
from jax.experimental.pallas import tpu_sc as plsc

N_NODES = 10000
RANK = 256
N_EDGES = 160000
N_RELS = 100
HALF = 128
NSUB = 16
CH = 40
E_PER_TILE = N_EDGES // NSUB
N_CHUNKS = E_PER_TILE // CH
N_PAD = 10240
ROWS_PER_TILE = N_PAD // NSUB
RCH = 32
PQ_HALF = 2 * N_RELS * HALF
QOFF = N_RELS * HALF
SLOPE = (1.0 / 8.0 + 1.0 / 3.0) / 2.0

_GDN = lax.GatherDimensionNumbers(
    offset_dims=(), collapsed_slice_dims=(0,), start_index_map=(0,))


def _vperm(v, idx):
    return lax.gather(v, idx[:, None], _GDN, (1,),
                      mode=lax.GatherScatterMode.PROMISE_IN_BOUNDS)


def _sc_body(xh, srcb, dst, et, pqf, normf, out,
             pq_v, norm_v, outbuf,
             xb0, xb1, xb2,
             sv0, sv1, sv2, sv3, sv4, sv5,
             dv0, dv1, dv2, dv3, dv4, dv5,
             ev0, ev1, ev2, ev3, ev4, ev5,
             acc,
             gsem0, gsem1, gsem2, ssem0, ssem1, ssem2,
             isem0, isem1, isem2, isem3, isem4, isem5):
    xbufs = (xb0, xb1, xb2)
    srcv = (sv0, sv1, sv2, sv3, sv4, sv5)
    dstv = (dv0, dv1, dv2, dv3, dv4, dv5)
    etv = (ev0, ev1, ev2, ev3, ev4, ev5)
    gsem = (gsem0, gsem1, gsem2)
    ssem = (ssem0, ssem1, ssem2)
    isem = (isem0, isem1, isem2, isem3, isem4, isem5)

    c = lax.axis_index("c")
    s = lax.axis_index("s")
    iota = lax.iota(jnp.int32, 16)
    swap_iota = iota - 2 * (iota % 2) + 1

    pltpu.sync_copy(pqf.at[pl.ds(c * PQ_HALF, PQ_HALF)], pq_v)
    r0 = s * ROWS_PER_TILE
    pltpu.sync_copy(normf.at[pl.ds(r0, ROWS_PER_TILE)],
                    norm_v.at[pl.ds(0, ROWS_PER_TILE)])

    zero = jnp.zeros((16,), jnp.float32)

    @pl.loop(0, RCH)
    def _zero_rows(i):
        row = outbuf.at[i]
        for t in range(HALF // 16):
            row[pl.ds(t * 16, 16)] = zero

    for kk in range(ROWS_PER_TILE // RCH):
        pltpu.sync_copy(outbuf, acc.at[pl.ds(r0 + kk * RCH, RCH)])
    plsc.subcore_barrier()

    e0 = s * E_PER_TILE

    def idx_triple(j, k):
        base = e0 + k * CH
        return ((srcb.at[pl.ds(c * N_EDGES + base, CH)], srcv[j], isem[j]),
                (dst.at[pl.ds(base, CH)], dstv[j], isem[j]),
                (et.at[pl.ds(base, CH)], etv[j].at[pl.ds(0, CH)], isem[j]))

    def idx_copy(j, k):
        for a, b, sm in idx_triple(j, k):
            pltpu.async_copy(a, b, sm)

    def idx_wait(j, k):
        for a, b, sm in idx_triple(j, k):
            pltpu.make_async_copy(a, b, sm).wait()

    def gather_start(r, j):
        pltpu.async_copy(xh.at[srcv[j]], xbufs[r], gsem[r])

    def gather_wait(r, j):
        pltpu.make_async_copy(xh.at[srcv[j]], xbufs[r], gsem[r]).wait()

    def scatter_start(r, j):
        pltpu.async_copy(xbufs[r], acc.at[dstv[j]], ssem[r], add=True)

    def scatter_wait(r, j):
        pltpu.make_async_copy(xbufs[r], acc.at[dstv[j]], ssem[r]).wait()

    def compute(r, j):
        xb = xbufs[r]
        etr = etv[j]

        @plsc.parallel_loop(0, CH, unroll=2)
        def _edge(e):
            xr = xb.at[e]
            pbase = etr[pl.ds(e, 16)][0]
            for t in range(HALF // 16):
                o = t * 16
                xv = xr[pl.ds(o, 16)]
                xs = _vperm(xv, swap_iota)
                pv = pq_v[pl.ds(pbase + o, 16)]
                qv = pq_v[pl.ds(pbase + (QOFF + o), 16)]
                xr[pl.ds(o, 16)] = xv * pv + xs * qv

    def body(k, u, wait_prev=True, do_icopy=True, do_gather=True):
        r = u % 3
        gather_wait(r, u)
        compute(r, u)
        scatter_start(r, u)
        if wait_prev:
            scatter_wait((u + 2) % 3, (u + 5) % 6)
        if do_icopy:
            idx_copy((u + 4) % 6, k + 4)
        if do_gather:
            idx_wait((u + 2) % 6, k + 2)
            gather_start((u + 2) % 3, (u + 2) % 6)

    scope_edges = jax.named_scope("edge_phase")
    scope_edges.__enter__()
    for j in range(4):
        idx_copy(j, j)
    idx_wait(0, 0)
    gather_start(0, 0)
    idx_wait(1, 1)
    gather_start(1, 1)

    body(0, 0, wait_prev=False)
    for k in range(1, 6):
        body(k, k)

    @pl.loop(0, (N_CHUNKS - 10) // 6)
    def _main(kk):
        kb = 6 + kk * 6
        for u in range(6):
            body(kb + u, u)

    for k in range(N_CHUNKS - 4, N_CHUNKS):
        body(k, k % 6, do_icopy=False, do_gather=(k + 2 < N_CHUNKS))

    scatter_wait((N_CHUNKS - 1) % 3, (N_CHUNKS - 1) % 6)
    scope_edges.__exit__(None, None, None)
    plsc.subcore_barrier()

    scope_fin = jax.named_scope("finalize_phase")
    scope_fin.__enter__()
    for kk in range(ROWS_PER_TILE // RCH):
        rbase = r0 + kk * RCH
        pltpu.sync_copy(acc.at[pl.ds(rbase, RCH)], outbuf)

        @pl.loop(0, RCH)
        def _post(i):
            row = outbuf.at[i]
            nrm = norm_v[pl.ds(kk * RCH + i, 16)][0]
            for t in range(HALF // 16):
                o = t * 16
                v = row[pl.ds(o, 16)] * nrm
                row[pl.ds(o, 16)] = jnp.where(v >= 0.0, v, v * SLOPE)

        pltpu.sync_copy(outbuf, out.at[pl.ds(rbase, RCH), pl.ds(c * HALF, HALF)])
    scope_fin.__exit__(None, None, None)


def kernel(x, norm, weight, edge_index, edge_type):
    x = x.astype(jnp.float32)
    src = edge_index[0].astype(jnp.int32)
    dst = edge_index[1].astype(jnp.int32)
    et = edge_type.astype(jnp.int32) * HALF

    w4 = weight.astype(jnp.float32).reshape(N_RELS, RANK // 2, 2, 2)
    p = jnp.stack([w4[:, :, 0, 0], w4[:, :, 1, 1]], -1).reshape(N_RELS, RANK)
    q = jnp.stack([w4[:, :, 1, 0], w4[:, :, 0, 1]], -1).reshape(N_RELS, RANK)
    ph = p.reshape(N_RELS, 2, HALF).transpose(1, 0, 2)
    qh = q.reshape(N_RELS, 2, HALF).transpose(1, 0, 2)
    pqf = jnp.concatenate([ph, qh], axis=1).reshape(-1)

    xh = x.reshape(N_NODES, 2, HALF).transpose(1, 0, 2).reshape(2 * N_NODES, HALF)
    srcb = jnp.concatenate([src, src + N_NODES])
    normf = jnp.pad(norm.astype(jnp.float32).reshape(-1), (0, N_PAD - N_NODES))

    mesh = plsc.VectorSubcoreMesh(core_axis_name="c", subcore_axis_name="s")
    dma = pltpu.SemaphoreType.DMA
    run = pl.kernel(
        _sc_body,
        out_type=jax.ShapeDtypeStruct((N_PAD, RANK), jnp.float32),
        mesh=mesh,
        scratch_types=(
            [pltpu.VMEM((PQ_HALF,), jnp.float32),
             pltpu.VMEM((ROWS_PER_TILE + 16,), jnp.float32),
             pltpu.VMEM((RCH, HALF), jnp.float32)]
            + [pltpu.VMEM((CH, HALF), jnp.float32)] * 3
            + [pltpu.VMEM((CH,), jnp.int32)] * 6
            + [pltpu.VMEM((CH,), jnp.int32)] * 6
            + [pltpu.VMEM((CH + 16,), jnp.int32)] * 6
            + [pltpu.VMEM_SHARED((N_PAD, HALF), jnp.float32)]
            + [dma] * 12
        ),
    )
    return run(xh, srcb, dst, et, pqf, normf)[:N_NODES]


# speedup vs baseline: 1.4231x; 1.3968x over previous
import jax
import jax.numpy as jnp
from jax import lax
from jax.experimental import pallas as pl
from jax.experimental.pallas import tpu as pltpu
from jax.experimental.pallas import tpu_sc as plsc

N_NODES = 10000
RANK = 256
N_EDGES = 160000
N_RELS = 100
HALF = 128
NSUB = 16
CH = 80
E_PER_TILE = N_EDGES // NSUB
N_CHUNKS = E_PER_TILE // CH
N_PAD = 10240
ROWS_PER_TILE = N_PAD // NSUB
RCH = 16
PQ_HALF = N_RELS * HALF
SLOPE = (1.0 / 8.0 + 1.0 / 3.0) / 2.0

_GDN = lax.GatherDimensionNumbers(
    offset_dims=(), collapsed_slice_dims=(0,), start_index_map=(0,))


def _vperm(v, idx):
    return lax.gather(v, idx[:, None], _GDN, (1,),
                      mode=lax.GatherScatterMode.PROMISE_IN_BOUNDS)


def _sc_body(xh, srcb, dst, et, pqf, normf, out,
             pq_v, norm_v, outbuf,
             xb0, xb1, xb2,
             sv0, sv1, sv2, sv3, sv4, sv5,
             dv0, dv1, dv2, dv3, dv4, dv5,
             ev0, ev1, ev2, ev3, ev4, ev5,
             acc,
             gsem0, gsem1, gsem2, ssem0, ssem1, ssem2,
             isem0, isem1, isem2, isem3, isem4, isem5,
             psem, zsem):
    xbufs = (xb0, xb1, xb2)
    srcv = (sv0, sv1, sv2, sv3, sv4, sv5)
    dstv = (dv0, dv1, dv2, dv3, dv4, dv5)
    etv = (ev0, ev1, ev2, ev3, ev4, ev5)
    gsem = (gsem0, gsem1, gsem2)
    ssem = (ssem0, ssem1, ssem2)
    isem = (isem0, isem1, isem2, isem3, isem4, isem5)

    c = lax.axis_index("c")
    s = lax.axis_index("s")
    iota = lax.iota(jnp.int32, 16)
    swap_iota = iota - 2 * (iota % 2) + 1
    r0 = s * ROWS_PER_TILE
    e0 = s * E_PER_TILE

    pq_load = pltpu.async_copy(pqf.at[pl.ds(c * PQ_HALF, PQ_HALF)], pq_v, psem)
    nrm_load = pltpu.async_copy(normf.at[pl.ds(r0, ROWS_PER_TILE)],
                                norm_v.at[pl.ds(0, ROWS_PER_TILE)], psem)

    zero = jnp.zeros((16,), jnp.float32)

    @pl.loop(0, RCH)
    def _zero_rows(i):
        row = outbuf.at[i]
        for t in range(HALF // 16):
            row[pl.ds(t * 16, 16)] = zero

    for kk in range(ROWS_PER_TILE // RCH):
        pltpu.async_copy(outbuf, acc.at[pl.ds(r0 + kk * RCH, RCH)], zsem)


    def idx_triple(j, k):
        base = e0 + k * CH
        return ((srcb.at[pl.ds(c * N_EDGES + base, CH)], srcv[j], isem[j]),
                (dst.at[pl.ds(base, CH)], dstv[j], isem[j]),
                (et.at[pl.ds(base, CH)], etv[j].at[pl.ds(0, CH)], isem[j]))

    def idx_copy(j, k):
        for a, b, sm in idx_triple(j, k):
            pltpu.async_copy(a, b, sm)

    def idx_wait(j, k):
        for a, b, sm in idx_triple(j, k):
            pltpu.make_async_copy(a, b, sm).wait()

    def gather_start(r, j):
        pltpu.async_copy(xh.at[srcv[j]], xbufs[r], gsem[r])

    def gather_wait(r, j):
        pltpu.make_async_copy(xh.at[srcv[j]], xbufs[r], gsem[r]).wait()

    def scatter_start(r, j):
        pltpu.async_copy(xbufs[r], acc.at[dstv[j]], ssem[r], add=True)

    def scatter_wait(r, j):
        pltpu.make_async_copy(xbufs[r], acc.at[dstv[j]], ssem[r]).wait()

    def compute(r, j):
        xb = xbufs[r]
        etr = etv[j]

        @plsc.parallel_loop(0, CH, unroll=2)
        def _edge(e):
            xr = xb.at[e]
            pbase = etr[pl.ds(e, 16)][0]
            for t in range(HALF // 16):
                o = t * 16
                xv = xr[pl.ds(o, 16)]
                xs = _vperm(xv, swap_iota)
                w = pq_v[pl.ds(pbase + o, 16)]
                pv = lax.bitcast_convert_type(w & jnp.int32(-65536), jnp.float32)
                qv = lax.bitcast_convert_type(w << 16, jnp.float32)
                xr[pl.ds(o, 16)] = xv * pv + xs * qv

    def body(k, u, wait_prev=True, do_icopy=True, do_gather=True):
        r = u % 3
        gather_wait(r, u)
        compute(r, u)
        scatter_start(r, u)
        if wait_prev:
            scatter_wait((u + 2) % 3, (u + 5) % 6)
        if do_icopy:
            idx_copy((u + 4) % 6, k + 4)
        if do_gather:
            idx_wait((u + 2) % 6, k + 2)
            gather_start((u + 2) % 3, (u + 2) % 6)

    for j in range(4):
        idx_copy(j, j)
    idx_wait(0, 0)
    gather_start(0, 0)
    idx_wait(1, 1)
    gather_start(1, 1)

    pq_load.wait()
    nrm_load.wait()
    for kk in range(ROWS_PER_TILE // RCH):
        pltpu.make_async_copy(outbuf, acc.at[pl.ds(r0 + kk * RCH, RCH)],
                              zsem).wait()
    plsc.subcore_barrier()

    body(0, 0, wait_prev=False)
    for k in range(1, 6):
        body(k, k)

    n_main = (N_CHUNKS - 11) // 6

    @pl.loop(0, n_main)
    def _main(kk):
        kb = 6 + kk * 6
        for u in range(6):
            body(kb + u, u)

    for k in range(6 + 6 * n_main, N_CHUNKS):
        body(k, k % 6, do_icopy=(k + 4 < N_CHUNKS),
             do_gather=(k + 2 < N_CHUNKS))

    scatter_wait((N_CHUNKS - 1) % 3, (N_CHUNKS - 1) % 6)
    plsc.subcore_barrier()

    for kk in range(ROWS_PER_TILE // RCH):
        rbase = r0 + kk * RCH
        pltpu.sync_copy(acc.at[pl.ds(rbase, RCH)], outbuf)

        @pl.loop(0, RCH)
        def _post(i):
            row = outbuf.at[i]
            nrm = norm_v[pl.ds(kk * RCH + i, 16)][0]
            for t in range(HALF // 16):
                o = t * 16
                v = row[pl.ds(o, 16)] * nrm
                row[pl.ds(o, 16)] = jnp.where(v >= 0.0, v, v * SLOPE)

        pltpu.sync_copy(outbuf, out.at[pl.ds(rbase, RCH), pl.ds(c * HALF, HALF)])


def kernel(x, norm, weight, edge_index, edge_type):
    x = x.astype(jnp.float32)
    src = edge_index[0].astype(jnp.int32)
    dst = edge_index[1].astype(jnp.int32)
    et = edge_type.astype(jnp.int32) * HALF

    w4 = weight.astype(jnp.float32).reshape(N_RELS, RANK // 2, 2, 2)
    p = jnp.stack([w4[:, :, 0, 0], w4[:, :, 1, 1]], -1).reshape(N_RELS, RANK)
    q = jnp.stack([w4[:, :, 1, 0], w4[:, :, 0, 1]], -1).reshape(N_RELS, RANK)
    ph = p.reshape(N_RELS, 2, HALF).transpose(1, 0, 2)
    qh = q.reshape(N_RELS, 2, HALF).transpose(1, 0, 2)
    p16 = lax.bitcast_convert_type(ph.astype(jnp.bfloat16), jnp.uint16)
    q16 = lax.bitcast_convert_type(qh.astype(jnp.bfloat16), jnp.uint16)
    pqf = (p16.astype(jnp.int32) << 16 | q16.astype(jnp.int32)).reshape(-1)

    xh = x.reshape(N_NODES, 2, HALF).transpose(1, 0, 2).reshape(2 * N_NODES, HALF)
    srcb = jnp.concatenate([src, src + N_NODES])
    normf = jnp.pad(norm.astype(jnp.float32).reshape(-1), (0, N_PAD - N_NODES))

    mesh = plsc.VectorSubcoreMesh(core_axis_name="c", subcore_axis_name="s")
    dma = pltpu.SemaphoreType.DMA
    run = pl.kernel(
        _sc_body,
        out_type=jax.ShapeDtypeStruct((N_PAD, RANK), jnp.float32),
        mesh=mesh,
        scratch_types=(
            [pltpu.VMEM((PQ_HALF,), jnp.int32),
             pltpu.VMEM((ROWS_PER_TILE + 16,), jnp.float32),
             pltpu.VMEM((RCH, HALF), jnp.float32)]
            + [pltpu.VMEM((CH, HALF), jnp.float32)] * 3
            + [pltpu.VMEM((CH,), jnp.int32)] * 6
            + [pltpu.VMEM((CH,), jnp.int32)] * 6
            + [pltpu.VMEM((CH + 16,), jnp.int32)] * 6
            + [pltpu.VMEM_SHARED((N_PAD, HALF), jnp.float32)]
            + [dma] * 14
        ),
    )
    return run(xh, srcb, dst, et, pqf, normf)[:N_NODES]
